# baseline (speedup 1.0000x reference)
import functools

import jax
import jax.numpy as jnp
from jax import lax
from jax.experimental import pallas as pl
from jax.experimental.pallas import tpu as pltpu
from jax.experimental.pallas import tpu_sc as plsc

N_NODES = 10000
N_EDGES = 320000
D_FEAT = 128
D_HID = 16

NC = 2
NS = 16
NW = NC * NS
EPT = N_EDGES // NW
CHUNK = 128
NCHUNK = 80
EPAD = NCHUNK * CHUNK - EPT
TBL = 10240
DUMMY = N_NODES + 16
ZROWS = TBL // NS
ZSTEP = 128

@functools.cache
def _mesh():
    return plsc.VectorSubcoreMesh(
        core_axis_name="c", subcore_axis_name="s", num_cores=NC, num_subcores=NS
    )


def _zero_stripe(acc, zbuf, s):
    def zrow(i, _):
        zbuf[i, :] = jnp.zeros((16,), jnp.float32)
        return 0

    lax.fori_loop(0, ZSTEP, zrow, 0)

    def zcp(i, _):
        pltpu.sync_copy(zbuf, acc.at[pl.ds(s * ZROWS + i * ZSTEP, ZSTEP)])
        return 0

    lax.fori_loop(0, ZROWS // ZSTEP, zcp, 0)


def _copy_out(acc, out_hbm, c, s):
    pltpu.sync_copy(
        acc.at[pl.ds(s * ZROWS, ZROWS)], out_hbm.at[c, pl.ds(s * ZROWS, ZROWS)]
    )


@functools.cache
def _sc_degree_kernel():
    return pl.kernel(
        _sc_degree_body,
        out_type=jax.ShapeDtypeStruct((NC, TBL, D_HID), jnp.float32),
        mesh=_mesh(),
        compiler_params=pltpu.CompilerParams(use_tc_tiling_on_sc=False),
        scratch_types=[
            pltpu.VMEM_SHARED((TBL, D_HID), jnp.float32),
            pltpu.VMEM((NCHUNK, CHUNK), jnp.int32),
            pltpu.VMEM((CHUNK, D_HID), jnp.float32),
            pltpu.VMEM((ZSTEP, D_HID), jnp.float32),
            pltpu.SemaphoreType.DMA,
        ],
    )


DEG_WIN = 8


def _sc_degree_body(col_hbm, out_hbm, acc, cidx, ones_b, zbuf, ssem):
    c = lax.axis_index("c")
    s = lax.axis_index("s")
    wid = c * NS + s
    _zero_stripe(acc, zbuf, s)

    def orow(i, _):
        ones_b[i, :] = jnp.ones((16,), jnp.float32)
        return 0

    lax.fori_loop(0, CHUNK, orow, 0)
    pltpu.sync_copy(col_hbm.at[wid], cidx)
    plsc.subcore_barrier()

    def step(j, _):
        @pl.when(j >= DEG_WIN)
        def _():
            pltpu.make_async_copy(ones_b, acc.at[cidx.at[j - DEG_WIN]], ssem).wait()

        pltpu.async_copy(ones_b, acc.at[cidx.at[j]], ssem, add=True)
        return 0

    lax.fori_loop(0, NCHUNK, step, 0)

    def drain(j, _):
        pltpu.make_async_copy(ones_b, acc.at[cidx.at[NCHUNK - DEG_WIN + j]], ssem).wait()
        return 0

    lax.fori_loop(0, DEG_WIN, drain, 0)
    plsc.subcore_barrier()
    _copy_out(acc, out_hbm, c, s)


@functools.cache
def _sc_propagate_kernel():
    return pl.kernel(
        _sc_propagate_body,
        out_type=jax.ShapeDtypeStruct((NC, TBL, D_HID), jnp.float32),
        mesh=_mesh(),
        compiler_params=pltpu.CompilerParams(use_tc_tiling_on_sc=False),
        scratch_types=[
            pltpu.VMEM_SHARED((TBL, D_HID), jnp.float32),
            pltpu.VMEM((NCHUNK, CHUNK), jnp.int32),
            pltpu.VMEM((NCHUNK, CHUNK), jnp.int32),
            pltpu.VMEM((NBUF, CHUNK, D_HID), jnp.float32),
            pltpu.VMEM((ZSTEP, D_HID), jnp.float32),
            pltpu.SemaphoreType.DMA,
            pltpu.SemaphoreType.DMA,
        ],
    )


NBUF = 40
AHEAD = NBUF // 2


def _sc_propagate_body(
    row_hbm, col_hbm, tbl_hbm, out_hbm, acc, ridx, cidx, bufs, zbuf, gsem, ssem
):
    c = lax.axis_index("c")
    s = lax.axis_index("s")
    wid = c * NS + s
    _zero_stripe(acc, zbuf, s)
    pltpu.sync_copy(row_hbm.at[wid], ridx)
    pltpu.sync_copy(col_hbm.at[wid], cidx)
    plsc.subcore_barrier()

    for j in range(AHEAD):
        pltpu.async_copy(tbl_hbm.at[ridx.at[j]], bufs.at[j], gsem)

    def outer(i, _):
        for bb in range(NBUF):
            j = NBUF * i + bb
            pltpu.make_async_copy(tbl_hbm.at[ridx.at[j]], bufs.at[bb], gsem).wait()
            pltpu.async_copy(bufs.at[bb], acc.at[cidx.at[j]], ssem, add=True)

            @pl.when(j >= AHEAD)
            def _():
                pltpu.make_async_copy(
                    bufs.at[(bb + AHEAD) % NBUF], acc.at[cidx.at[j - AHEAD]], ssem
                ).wait()

            @pl.when(j + AHEAD < NCHUNK)
            def _():
                pltpu.async_copy(
                    tbl_hbm.at[ridx.at[j + AHEAD]], bufs.at[(bb + AHEAD) % NBUF], gsem
                )
        return 0

    lax.fori_loop(0, NCHUNK // NBUF, outer, 0)
    for k in range(AHEAD):
        j = NCHUNK - AHEAD + k
        pltpu.make_async_copy(
            bufs.at[j % NBUF], acc.at[cidx.at[j]], ssem
        ).wait()
    plsc.subcore_barrier()
    _copy_out(acc, out_hbm, c, s)


ROWS_B = 2000
GRID = N_NODES // ROWS_B


def _tc_prep_body(x_ref, w_ref, a0_ref, y1_ref):
    xw = jnp.dot(x_ref[...], w_ref[...], preferred_element_type=jnp.float32)
    a0_ref[...] = xw[:, :D_HID]
    y1_ref[...] = xw[:, D_HID:]


def _tc_prep(x, w1c):
    return pl.pallas_call(
        _tc_prep_body,
        grid=(GRID,),
        in_specs=[
            pl.BlockSpec((ROWS_B, D_FEAT), lambda i: (i, 0)),
            pl.BlockSpec((D_FEAT, 2 * D_HID), lambda i: (0, 0)),
        ],
        out_specs=[pl.BlockSpec((ROWS_B, D_HID), lambda i: (i, 0))] * 2,
        out_shape=[jax.ShapeDtypeStruct((N_NODES, D_HID), jnp.float32)] * 2,
    )(x, w1c)


def _tc_mid_body(hin_ref, w_ref, c0_ref, z1_ref):
    h = jnp.maximum(hin_ref[...], 0.0)
    hw = jnp.dot(h, w_ref[...], preferred_element_type=jnp.float32)
    c0_ref[...] = hw[:, :D_HID]
    z1_ref[...] = hw[:, D_HID:]


def _tc_mid(hin, w2c):
    return pl.pallas_call(
        _tc_mid_body,
        grid=(GRID,),
        in_specs=[
            pl.BlockSpec((ROWS_B, D_HID), lambda i: (i, 0)),
            pl.BlockSpec((D_HID, 2 * D_HID), lambda i: (0, 0)),
        ],
        out_specs=[pl.BlockSpec((ROWS_B, D_HID), lambda i: (i, 0))] * 2,
        out_shape=[jax.ShapeDtypeStruct((N_NODES, D_HID), jnp.float32)] * 2,
    )(hin, w2c)


def _tc_final_body(o_ref, out_ref):
    o = o_ref[...]
    m = jnp.max(o, axis=1, keepdims=True)
    e = jnp.exp(o - m)
    out_ref[...] = (o - m) - jnp.log(jnp.sum(e, axis=1, keepdims=True))


def _tc_final(o):
    return pl.pallas_call(
        _tc_final_body,
        grid=(GRID,),
        in_specs=[pl.BlockSpec((ROWS_B, D_HID), lambda i: (i, 0))],
        out_specs=pl.BlockSpec((ROWS_B, D_HID), lambda i: (i, 0)),
        out_shape=jax.ShapeDtypeStruct((N_NODES, D_HID), jnp.float32),
    )(o)


def kernel(x, edge_index, W1_0, W1_1, b1, W2_0, W2_1, b2):
    ei = edge_index.astype(jnp.int32)
    row2 = ei[0].reshape(NW, EPT)
    col2 = ei[1].reshape(NW, EPT)
    rowp = jnp.concatenate(
        [row2, jnp.zeros((NW, EPAD), jnp.int32)], axis=1
    ).reshape(NW, NCHUNK, CHUNK)
    colp = jnp.concatenate(
        [col2, jnp.full((NW, EPAD), DUMMY, jnp.int32)], axis=1
    ).reshape(NW, NCHUNK, CHUNK)
    degp = _sc_degree_kernel()(colp)
    w1c = jnp.concatenate([W1_0, W1_1], axis=1)
    a0, y1 = _tc_prep(x, w1c)
    deg16 = degp[0, :N_NODES] + degp[1, :N_NODES]
    dis16 = jnp.where(deg16 > 0, lax.rsqrt(deg16), 0.0)
    ys = dis16 * y1
    pparts = _sc_propagate_kernel()(rowp, colp, ys)
    hin = a0 + dis16 * (pparts[0, :N_NODES] + pparts[1, :N_NODES]) + b1
    w2c = jnp.concatenate([W2_0, W2_1], axis=1)
    c0, z1 = _tc_mid(hin, w2c)
    zs = dis16 * z1
    qparts = _sc_propagate_kernel()(rowp, colp, zs)
    o = c0 + dis16 * (qparts[0, :N_NODES] + qparts[1, :N_NODES]) + b2
    return _tc_final(o)
